# baseline (speedup 1.0000x reference)
import jax
import jax.numpy as jnp
from jax.experimental import pallas as pl
from jax.experimental.pallas import tpu as pltpu

BLOCK_B = 2048


def _policy_body(x_ref, w_ref, b_ref, o_ref):
    logits = jax.lax.dot_general(
        w_ref[...], x_ref[...],
        dimension_numbers=(((0,), (0,)), ((), ())),
        preferred_element_type=jnp.float32,
    ) + b_ref[...]
    m = jnp.max(logits, axis=0, keepdims=True)
    s = jnp.sum(jnp.exp(logits - m), axis=0, keepdims=True)
    o_ref[...] = jnp.exp(logits - (m + jnp.log(s)))


def kernel(state, W, b):
    B, D = state.shape
    A = W.shape[1]
    xT = state.T
    bc = b.reshape(A, 1)
    probsT = pl.pallas_call(
        _policy_body,
        grid=(B // BLOCK_B,),
        in_specs=[
            pl.BlockSpec((D, BLOCK_B), lambda i: (0, i)),
            pl.BlockSpec((D, A), lambda i: (0, 0)),
            pl.BlockSpec((A, 1), lambda i: (0, 0)),
        ],
        out_specs=pl.BlockSpec((A, BLOCK_B), lambda i: (0, i)),
        out_shape=jax.ShapeDtypeStruct((A, B), jnp.float32),
        compiler_params=pltpu.CompilerParams(
            dimension_semantics=("parallel",),
        ),
    )(xT, W, bc)
    return probsT.T


# speedup vs baseline: 1.0464x; 1.0464x over previous
import jax
import jax.numpy as jnp
from jax.experimental import pallas as pl
from jax.experimental.pallas import tpu as pltpu

BLOCK_B = 2048


def _policy_body(x_ref, w_ref, b_ref, o_ref):
    i = pl.program_id(0)
    x_blk = x_ref[:, pl.ds(i * BLOCK_B, BLOCK_B)]
    logits = jax.lax.dot_general(
        w_ref[...], x_blk,
        dimension_numbers=(((0,), (0,)), ((), ())),
        preferred_element_type=jnp.float32,
    ) + b_ref[...]
    m = jnp.max(logits, axis=0, keepdims=True)
    e = jnp.exp(logits - m)
    r = 1.0 / jnp.sum(e, axis=0, keepdims=True)
    o_ref[...] = e * r


def kernel(state, W, b):
    B, D = state.shape
    A = W.shape[1]
    xT = state.T
    bc = b.reshape(A, 1)
    probsT = pl.pallas_call(
        _policy_body,
        grid=(B // BLOCK_B,),
        in_specs=[
            pl.BlockSpec((D, B), lambda i: (0, 0)),
            pl.BlockSpec((D, A), lambda i: (0, 0)),
            pl.BlockSpec((A, 1), lambda i: (0, 0)),
        ],
        out_specs=pl.BlockSpec((A, BLOCK_B), lambda i: (0, i)),
        out_shape=jax.ShapeDtypeStruct((A, B), jnp.float32),
        compiler_params=pltpu.CompilerParams(
            dimension_semantics=("parallel",),
        ),
    )(xT, W, bc)
    return probsT.T


# speedup vs baseline: 1.0671x; 1.0197x over previous
import jax
import jax.numpy as jnp
from jax.experimental import pallas as pl
from jax.experimental.pallas import tpu as pltpu

BLOCK_B = 2048


def _policy_body(x_ref, w_ref, b_ref, o_ref):
    logits = jax.lax.dot_general(
        w_ref[...], x_ref[...],
        dimension_numbers=(((0,), (0,)), ((), ())),
        preferred_element_type=jnp.float32,
    ) + b_ref[...]
    m = jnp.max(logits, axis=0, keepdims=True)
    e = jnp.exp(logits - m)
    r = 1.0 / jnp.sum(e, axis=0, keepdims=True)
    o_ref[...] = e * r


def kernel(state, W, b):
    B, D = state.shape
    A = W.shape[1]
    xT = state.T
    bc = b.reshape(A, 1)
    probsT = pl.pallas_call(
        _policy_body,
        grid=(B // BLOCK_B,),
        in_specs=[
            pl.BlockSpec((D, BLOCK_B), lambda i: (0, i)),
            pl.BlockSpec((D, A), lambda i: (0, 0)),
            pl.BlockSpec((A, 1), lambda i: (0, 0)),
        ],
        out_specs=pl.BlockSpec((A, BLOCK_B), lambda i: (0, i)),
        out_shape=jax.ShapeDtypeStruct((A, B), jnp.float32),
        compiler_params=pltpu.CompilerParams(
            dimension_semantics=("parallel",),
        ),
    )(xT, W, bc)
    return probsT.T
